# baseline (speedup 1.0000x reference)
import functools

import jax
import jax.numpy as jnp
from jax import lax
from jax.experimental import pallas as pl
from jax.experimental.pallas import tpu as pltpu
from jax.experimental.pallas import tpu_sc as plsc

_B = 16384
_L = 50
_D = 16
_ST = 5
_OUT = 64

_NC, _NS = 2, 16
_NW = _NC * _NS
_RPW = _B // _NW
_CH = 32
_NCHUNK = _RPW // _CH
_LP = 56
_GI = _CH * _LP
_IDP = 128


def _sc_pool(ids, table, zrs, seg):
    mesh = plsc.VectorSubcoreMesh(core_axis_name="c", subcore_axis_name="s")

    @functools.partial(
        pl.kernel,
        mesh=mesh,
        compiler_params=pltpu.CompilerParams(use_tc_tiling_on_sc=False),
        out_type=jax.ShapeDtypeStruct((_B, _D), jnp.float32),
        scratch_types=[
            pltpu.VMEM((_GI,), jnp.int32),
            pltpu.VMEM((_GI,), jnp.int32),
            pltpu.VMEM((_GI,), jnp.int32),
            pltpu.VMEM((_GI, _D), jnp.float32),
            pltpu.VMEM((_GI, _D), jnp.float32),
            pltpu.SemaphoreType.DMA,
            pltpu.SemaphoreType.DMA,
            pltpu.SemaphoreType.DMA,
            pltpu.SemaphoreType.DMA,
            pltpu.VMEM_SHARED((_NS * _CH, _D), jnp.float32),
        ],
    )
    def body(ids_hbm, table_hbm, z_hbm, seg_hbm, out_hbm,
             idx_a, idx_b, seg_v, rows_a, rows_b,
             isem_a, isem_b, gsem_a, gsem_b, pool_s):
        sid = lax.axis_index("s")
        wid = sid * _NC + lax.axis_index("c")
        pltpu.sync_copy(seg_hbm.at[pl.ds(sid * _GI, _GI)], seg_v)
        bufs = ((idx_a, rows_a, isem_a, gsem_a),
                (idx_b, rows_b, isem_b, gsem_b))

        def fire_idx(c):
            idx, _, isem, _ = bufs[c % 2]
            base = wid * _RPW + c * _CH

            @pl.loop(0, _CH)
            def _(p):
                pltpu.async_copy(ids_hbm.at[base + p, pl.ds(0, _LP)],
                                 idx.at[pl.ds(p * _LP, _LP)], isem)

        def fire_gather(c):
            idx, rows, isem, gsem = bufs[c % 2]
            pltpu.make_async_copy(seg_hbm.at[pl.ds(0, _GI)], idx, isem).wait()
            return pltpu.async_copy(table_hbm.at[idx], rows, gsem)

        fire_idx(0)
        gathers = [fire_gather(0)] + [None] * (_NCHUNK - 1)
        for c in range(_NCHUNK):
            _, rows, _, _ = bufs[c % 2]
            if c + 1 < _NCHUNK:
                fire_idx(c + 1)
            gathers[c].wait()
            if c + 1 < _NCHUNK:
                gathers[c + 1] = fire_gather(c + 1)
            base = wid * _RPW + c * _CH
            pltpu.sync_copy(z_hbm, pool_s.at[pl.ds(sid * _CH, _CH)])
            pltpu.sync_copy(rows, pool_s.at[seg_v], add=True)
            pltpu.sync_copy(pool_s.at[pl.ds(sid * _CH, _CH)],
                            out_hbm.at[pl.ds(base, _CH)])

    return body(ids, table, zrs, seg)


def _tc_head(ids, sums, stats, w, b):
    rows = 2048
    grid = (_B // rows,)

    def body(ids_ref, sums_ref, stats_ref, w_ref, b_ref, out_ref):
        idv = ids_ref[...]
        cnt = jnp.sum((idv != 0).astype(jnp.float32), axis=1)
        inv = 1.0 / jnp.maximum(cnt, 1.0)
        wm = w_ref[...]
        accp = lax.dot_general(wm[:, :_D], sums_ref[...],
                               (((1,), (1,)), ((), ())),
                               preferred_element_type=jnp.float32)
        accs = lax.dot_general(wm[:, _D:], stats_ref[...],
                               (((1,), (1,)), ((), ())),
                               preferred_element_type=jnp.float32)
        acc = accp * inv[None, :] + accs + b_ref[...]
        out_ref[...] = jnp.maximum(acc, 0.0)

    return pl.pallas_call(
        body,
        grid=grid,
        in_specs=[
            pl.BlockSpec((rows, _L), lambda i: (i, 0)),
            pl.BlockSpec((rows, _D), lambda i: (i, 0)),
            pl.BlockSpec((rows, _ST), lambda i: (i, 0)),
            pl.BlockSpec((_OUT, _D + _ST), lambda i: (0, 0)),
            pl.BlockSpec((_OUT, 1), lambda i: (0, 0)),
        ],
        out_specs=pl.BlockSpec((_OUT, rows), lambda i: (0, i)),
        out_shape=jax.ShapeDtypeStruct((_OUT, _B), jnp.float32),
    )(ids, sums, stats, w, b.reshape(_OUT, 1))


def kernel(skill_ids, stats, skill_emb, proj_W, proj_b):
    ids_pad = jnp.pad(skill_ids, ((0, 0), (0, _IDP - _L)))
    seg = (jnp.arange(_GI, dtype=jnp.int32) // _LP)[None, :] + (
        jnp.arange(_NS, dtype=jnp.int32) * _CH)[:, None]
    seg = seg.reshape(_NS * _GI)
    zrs = jnp.zeros((_CH, _D), jnp.float32)
    sums = _sc_pool(ids_pad, skill_emb, zrs, seg)
    out_t = _tc_head(skill_ids, sums, stats, proj_W, proj_b)
    return out_t.T


# speedup vs baseline: 4.0902x; 4.0902x over previous
import functools

import jax
import jax.numpy as jnp
from jax import lax
from jax.experimental import pallas as pl
from jax.experimental.pallas import tpu as pltpu
from jax.experimental.pallas import tpu_sc as plsc

_B = 16384
_L = 50
_D = 16
_ST = 5
_OUT = 64

_NC, _NS = 2, 16
_NW = _NC * _NS
_RPW = _B // _NW
_CH = 64
_NCHUNK = _RPW // _CH
_GI = _CH * _L


def _sc_pool(ids, table, zrs, seg):
    mesh = plsc.VectorSubcoreMesh(core_axis_name="c", subcore_axis_name="s")

    @functools.partial(
        pl.kernel,
        mesh=mesh,
        compiler_params=pltpu.CompilerParams(use_tc_tiling_on_sc=False),
        out_type=jax.ShapeDtypeStruct((_B, _D), jnp.float32),
        scratch_types=[
            pltpu.VMEM((_GI,), jnp.int32),
            pltpu.VMEM((_GI,), jnp.int32),
            pltpu.VMEM((_GI,), jnp.int32),
            pltpu.VMEM((_GI, _D), jnp.float32),
            pltpu.VMEM((_GI, _D), jnp.float32),
            pltpu.SemaphoreType.DMA,
            pltpu.SemaphoreType.DMA,
            pltpu.VMEM_SHARED((_NS * _CH, _D), jnp.float32),
        ],
    )
    def body(ids_hbm, table_hbm, z_hbm, seg_hbm, out_hbm,
             idx_a, idx_b, seg_v, rows_a, rows_b, gsem_a, gsem_b, pool_s):
        sid = lax.axis_index("s")
        wid = sid * _NC + lax.axis_index("c")
        pltpu.sync_copy(seg_hbm.at[pl.ds(sid * _GI, _GI)], seg_v)
        bufs = ((idx_a, rows_a, gsem_a), (idx_b, rows_b, gsem_b))

        def load_and_gather(c):
            idx, rows, gsem = bufs[c % 2]
            base = wid * _RPW + c * _CH
            pltpu.sync_copy(ids_hbm.at[pl.ds(base * _L, _GI)], idx)
            return pltpu.async_copy(table_hbm.at[idx], rows, gsem)

        gathers = [load_and_gather(0)] + [None] * (_NCHUNK - 1)
        for c in range(_NCHUNK):
            _, rows, _ = bufs[c % 2]
            if c + 1 < _NCHUNK:
                gathers[c + 1] = load_and_gather(c + 1)
            gathers[c].wait()
            base = wid * _RPW + c * _CH
            pltpu.sync_copy(z_hbm, pool_s.at[pl.ds(sid * _CH, _CH)])
            pltpu.sync_copy(rows, pool_s.at[seg_v], add=True)
            pltpu.sync_copy(pool_s.at[pl.ds(sid * _CH, _CH)],
                            out_hbm.at[pl.ds(base, _CH)])

    return body(ids, table, zrs, seg)


def _tc_head(ids, sums, stats, w, b):
    rows = 2048
    grid = (_B // rows,)

    def body(ids_ref, sums_ref, stats_ref, w_ref, b_ref, out_ref):
        idv = ids_ref[...]
        cnt = jnp.sum((idv != 0).astype(jnp.float32), axis=1)
        inv = 1.0 / jnp.maximum(cnt, 1.0)
        wm = w_ref[...]
        accp = lax.dot_general(wm[:, :_D], sums_ref[...],
                               (((1,), (1,)), ((), ())),
                               preferred_element_type=jnp.float32)
        accs = lax.dot_general(wm[:, _D:], stats_ref[...],
                               (((1,), (1,)), ((), ())),
                               preferred_element_type=jnp.float32)
        acc = accp * inv[None, :] + accs + b_ref[...]
        out_ref[...] = jnp.maximum(acc, 0.0)

    return pl.pallas_call(
        body,
        grid=grid,
        in_specs=[
            pl.BlockSpec((rows, _L), lambda i: (i, 0)),
            pl.BlockSpec((rows, _D), lambda i: (i, 0)),
            pl.BlockSpec((rows, _ST), lambda i: (i, 0)),
            pl.BlockSpec((_OUT, _D + _ST), lambda i: (0, 0)),
            pl.BlockSpec((_OUT, 1), lambda i: (0, 0)),
        ],
        out_specs=pl.BlockSpec((_OUT, rows), lambda i: (0, i)),
        out_shape=jax.ShapeDtypeStruct((_OUT, _B), jnp.float32),
    )(ids, sums, stats, w, b.reshape(_OUT, 1))


def kernel(skill_ids, stats, skill_emb, proj_W, proj_b):
    ids_flat = skill_ids.reshape(_B * _L)
    seg = (jnp.arange(_GI, dtype=jnp.int32) // _L)[None, :] + (
        jnp.arange(_NS, dtype=jnp.int32) * _CH)[:, None]
    seg = seg.reshape(_NS * _GI)
    zrs = jnp.zeros((_CH, _D), jnp.float32)
    sums = _sc_pool(ids_flat, skill_emb, zrs, seg)
    out_t = _tc_head(skill_ids, sums, stats, proj_W, proj_b)
    return out_t.T
